# baseline (speedup 1.0000x reference)
import functools

import jax
import jax.numpy as jnp
from jax import lax
from jax.experimental import pallas as pl
from jax.experimental.pallas import tpu as pltpu
from jax.experimental.pallas import tpu_sc as plsc


_BS = 512


def _add_body(in_ref, pos_ref, out_ref):
    out_ref[...] = in_ref[...] + pos_ref[...][None]


def _tc_kernel(inputs, pos_table, nb=None):
    batch, seq_len, dim = inputs.shape
    nb = batch if nb is None else nb
    grid = (seq_len // _BS,)
    return pl.pallas_call(
        _add_body,
        grid=grid,
        in_specs=[
            pl.BlockSpec((nb, _BS, dim), lambda s: (0, s, 0)),
            pl.BlockSpec((_BS, dim), lambda s: (s, 0)),
        ],
        out_specs=pl.BlockSpec((nb, _BS, dim), lambda s: (0, s, 0)),
        out_shape=jax.ShapeDtypeStruct((nb, seq_len, dim), inputs.dtype),
    )(inputs, pos_table)


_SC_R = 32


def _sc_body(batch, seq_len, dim, in_row_base, in_hbm, pos_hbm, out_hbm,
             iv0, iv1, pv0, pv1, si0, si1, sp0, sp1, so0, so1):
    ivs, pvs = (iv0, iv1), (pv0, pv1)
    sis, sps, sos = (si0, si1), (sp0, sp1), (so0, so1)
    nc = 2
    seq_per_w = seq_len // 32
    nchunks = seq_per_w // _SC_R
    wid = lax.axis_index("s") * nc + lax.axis_index("c")
    seq0 = wid * seq_per_w
    nvec = dim // 16
    nsteps = nchunks * batch

    def out_row0(s):
        c, b = divmod(s, batch)
        return b * seq_len + seq0 + c * _SC_R

    def in_row0(s):
        return in_row_base + out_row0(s)

    pos_h = [None] * nchunks
    in_h = [None] * nsteps
    out_h = [None] * nsteps
    pos_h[0] = pltpu.async_copy(pos_hbm.at[pl.ds(seq0, _SC_R)], pvs[0], sps[0])
    in_h[0] = pltpu.async_copy(
        in_hbm.at[pl.ds(in_row0(0), _SC_R)], ivs[0], sis[0])

    for s in range(nsteps):
        c, b = divmod(s, batch)
        if b == 0:
            pos_h[c].wait()
        in_h[s].wait()
        if s + 1 < nsteps:
            if s - 1 >= 0:
                out_h[s - 1].wait()
            in_h[s + 1] = pltpu.async_copy(
                in_hbm.at[pl.ds(in_row0(s + 1), _SC_R)],
                ivs[(s + 1) % 2], sis[(s + 1) % 2])
        if b == batch - 1 and c + 1 < nchunks:
            pos_h[c + 1] = pltpu.async_copy(
                pos_hbm.at[pl.ds(seq0 + (c + 1) * _SC_R, _SC_R)],
                pvs[(c + 1) % 2], sps[(c + 1) % 2])
        iv, pv = ivs[s % 2], pvs[c % 2]

        def row(r, rcarry, iv=iv, pv=pv):
            for j in range(nvec):
                plsc.addupdate(
                    iv.at[r, pl.ds(j * 16, 16)],
                    pv[r, pl.ds(j * 16, 16)],
                )
            return rcarry

        lax.fori_loop(0, _SC_R, row, 0)
        out_h[s] = pltpu.async_copy(
            iv, out_hbm.at[pl.ds(out_row0(s), _SC_R)], sos[s % 2])
    if nsteps >= 2:
        out_h[nsteps - 2].wait()
    out_h[nsteps - 1].wait()


def _sc_kernel(flat_inputs, pos_table, batch, in_row_base=0):
    dim = flat_inputs.shape[1]
    seq_len = pos_table.shape[0]
    mesh = plsc.VectorSubcoreMesh(core_axis_name="c", subcore_axis_name="s")
    k = pl.kernel(
        functools.partial(_sc_body, batch, seq_len, dim, in_row_base),
        mesh=mesh,
        out_type=jax.ShapeDtypeStruct((batch * seq_len, dim),
                                      flat_inputs.dtype),
        scratch_types=[
            pltpu.VMEM((_SC_R, dim), jnp.float32),
            pltpu.VMEM((_SC_R, dim), jnp.float32),
            pltpu.VMEM((_SC_R, dim), jnp.float32),
            pltpu.VMEM((_SC_R, dim), jnp.float32),
            pltpu.SemaphoreType.DMA,
            pltpu.SemaphoreType.DMA,
            pltpu.SemaphoreType.DMA,
            pltpu.SemaphoreType.DMA,
            pltpu.SemaphoreType.DMA,
            pltpu.SemaphoreType.DMA,
        ],
    )
    return k(flat_inputs, pos_table)


_SC_BATCHES = 1


def kernel(inputs, pos_table):
    batch, seq_len, dim = inputs.shape
    nb_tc = batch - _SC_BATCHES
    flat = inputs.reshape(batch * seq_len, dim)
    out_sc = _sc_kernel(flat, pos_table, _SC_BATCHES,
                        in_row_base=nb_tc * seq_len)
    out_tc = _tc_kernel(inputs, pos_table, nb=nb_tc)
    return jnp.concatenate(
        [out_tc, out_sc.reshape(_SC_BATCHES, seq_len, dim)], axis=0)


# speedup vs baseline: 2.2507x; 2.2507x over previous
import jax
import jax.numpy as jnp
from jax.experimental import pallas as pl


_BS = 512


def _add_body(in_ref, pos_ref, out_ref):
    out_ref[...] = in_ref[...] + pos_ref[...][None]


def kernel(inputs, pos_table):
    batch, seq_len, dim = inputs.shape
    grid = (seq_len // _BS,)
    return pl.pallas_call(
        _add_body,
        grid=grid,
        in_specs=[
            pl.BlockSpec((batch, _BS, dim), lambda s: (0, s, 0)),
            pl.BlockSpec((_BS, dim), lambda s: (s, 0)),
        ],
        out_specs=pl.BlockSpec((batch, _BS, dim), lambda s: (0, s, 0)),
        out_shape=jax.ShapeDtypeStruct((batch, seq_len, dim), inputs.dtype),
    )(inputs, pos_table)
